# baseline (speedup 1.0000x reference)
import functools

import jax
import jax.numpy as jnp
from jax import lax
from jax.experimental import pallas as pl
from jax.experimental.pallas import tpu as pltpu
from jax.experimental.pallas import tpu_sc as plsc

N_CLS = 1000
N_CTX = 16
CTX_DIM = 512
N_PROMPTS = 32
SEQ = 77
B = 4096
SUFFIX_LEN = SEQ - 1 - N_CTX

_info = plsc.get_sparse_core_info()
_NC = _info.num_cores
_NS = _info.num_subcores
_NW = _NC * _NS
_BPW = B // _NW

_mesh = plsc.VectorSubcoreMesh(core_axis_name="c", subcore_axis_name="s")


@functools.partial(
    pl.kernel,
    mesh=_mesh,
    out_type=jax.ShapeDtypeStruct((B, SEQ, CTX_DIM), jnp.float32),
    scratch_types=[
        pltpu.VMEM((_BPW * 8,), jnp.int32),
        pltpu.VMEM((_BPW * 8,), jnp.int32),
        pltpu.VMEM((1, SEQ, CTX_DIM), jnp.float32),
        pltpu.SemaphoreType.DMA,
    ],
    compiler_params=pltpu.CompilerParams(use_tc_tiling_on_sc=False),
)
def _sc_concat_gather(ctx_hbm, prefix_hbm, suffix_hbm, labels_hbm, match_hbm,
                      out_hbm, labels_v, match_v, rbuf, sem):
    wid = lax.axis_index("s") * _NC + lax.axis_index("c")
    base = wid * _BPW
    pltpu.sync_copy(labels_hbm.at[pl.ds(base * 8, _BPW * 8)], labels_v)
    pltpu.sync_copy(match_hbm.at[pl.ds(base * 8, _BPW * 8)], match_v)

    def body(r):
        lidx = labels_v.at[pl.ds(r * 8, 1)]
        midx = match_v.at[pl.ds(r * 8, 1)]
        cp = pltpu.async_copy(prefix_hbm.at[lidx], rbuf.at[:, pl.ds(0, 1)], sem)
        cc = pltpu.async_copy(ctx_hbm.at[midx], rbuf.at[:, pl.ds(1, N_CTX)], sem)
        cs = pltpu.async_copy(suffix_hbm.at[lidx],
                              rbuf.at[:, pl.ds(1 + N_CTX, SUFFIX_LEN)], sem)
        cp.wait()
        cc.wait()
        cs.wait()
        row = base + r
        pltpu.sync_copy(rbuf, out_hbm.at[pl.ds(row, 1)])

    pl.loop(0, _BPW)(body)


def kernel(ctx, token_prefix, token_suffix, labels, match_ids):
    labels8 = jnp.broadcast_to(labels.astype(jnp.int32)[:, None], (B, 8)).reshape(B * 8)
    match8 = jnp.broadcast_to(match_ids.astype(jnp.int32)[:, None], (B, 8)).reshape(B * 8)
    return _sc_concat_gather(ctx, token_prefix, token_suffix, labels8, match8)
